# baseline (speedup 1.0000x reference)
import jax
import jax.numpy as jnp
from jax import lax
from jax.experimental import pallas as pl
from jax.experimental.pallas import tpu as pltpu

N = 8192
K = 8
BLOCK = 256
SLAB = 8
PAD_D = 8
LANES = 128
NCOL = N // LANES


def _cmp_full(av, ai, bv, bi):
    if av is None:
        return bv, bi, av, ai
    if bv is None:
        return av, ai, bv, bi
    c = av <= bv
    return (jnp.minimum(av, bv), jnp.where(c, ai, bi),
            jnp.maximum(av, bv), jnp.where(c, bi, ai))


def _cmp_lo(av, ai, bv, bi):
    if av is None:
        return bv, bi
    if bv is None:
        return av, ai
    c = av <= bv
    return jnp.minimum(av, bv), jnp.where(c, ai, bi)


def _sort_bitonic(vals, idxs, need):
    n = len(vals)
    if n == 1:
        return vals, idxs
    half = n // 2
    lov = [None] * half
    loi = [None] * half
    if need > half:
        hiv = [None] * half
        hii = [None] * half
        for i in range(half):
            lov[i], loi[i], hiv[i], hii[i] = _cmp_full(
                vals[i], idxs[i], vals[i + half], idxs[i + half])
        sl_v, sl_i = _sort_bitonic(lov, loi, half)
        sh_v, sh_i = _sort_bitonic(hiv, hii, need - half)
        return sl_v + sh_v, sl_i + sh_i
    for i in range(half):
        lov[i], loi[i] = _cmp_lo(vals[i], idxs[i], vals[i + half], idxs[i + half])
    return _sort_bitonic(lov, loi, need)


def _merge_sorted(av, ai, bv, bi, keep):
    tot = len(av) + len(bv)
    p = 1
    while p < tot:
        p *= 2
    pad = p - tot
    seq_v = list(av) + [None] * pad + list(bv[::-1])
    seq_i = list(ai) + [None] * pad + list(bi[::-1])
    need = min(keep, tot)
    rv, ri = _sort_bitonic(seq_v, seq_i, need)
    return rv[:need], ri[:need]


def _tree_top9(load_col, lane):
    def build(lo, hi):
        if hi - lo == 1:
            return [load_col(lo)], [lane + lo * LANES]
        mid = (lo + hi) // 2
        av, ai = build(lo, mid)
        bv, bi = build(mid, hi)
        return _merge_sorted(av, ai, bv, bi, K + 1)
    return build(0, NCOL)


def _knn_block_kernel(q_ref, k_ref, radii_ref, idx_ref, d2_ref):
    q = q_ref[...]
    kt = k_ref[...]
    qq = jnp.sum(q * q, axis=1, keepdims=True)
    kk = jnp.sum(kt * kt, axis=1)[None, :]
    qk = lax.dot_general(q, kt, (((1,), (1,)), ((), ())),
                         preferred_element_type=jnp.float32)
    d2_ref[...] = jnp.maximum(qq + kk - 2.0 * qk, 0.0)

    lane = lax.broadcasted_iota(jnp.int32, (SLAB, LANES), 1)

    def body(i, carry):
        r0 = i * SLAB

        def load_col(c):
            return d2_ref[pl.ds(r0, SLAB), pl.ds(c * LANES, LANES)]

        lv, li = _tree_top9(load_col, lane)
        lv = list(lv)

        radii_acc = jnp.zeros((SLAB, 1), dtype=jnp.float32)
        idx_cols = []
        for r in range(K + 1):
            mv = lv[0]
            for t in range(1, r + 1):
                mv = jnp.minimum(mv, lv[t])
            m = jnp.min(mv, axis=1, keepdims=True)
            ai = jnp.full((SLAB, LANES), N, dtype=jnp.int32)
            for t in range(r + 1):
                ai = jnp.where(lv[t] == m, jnp.minimum(ai, li[t]), ai)
            am = jnp.min(ai, axis=1, keepdims=True)
            idx_cols.append(am)
            if r > 0:
                radii_acc = radii_acc + jnp.sqrt(jnp.maximum(m, 1e-12))
            if r < K:
                for t in range(r + 1):
                    lv[t] = jnp.where(li[t] == am, jnp.float32(jnp.inf), lv[t])
        radii_ref[pl.ds(r0, SLAB), :] = radii_acc * (1.0 / K)
        idx_ref[pl.ds(r0, SLAB), :] = jnp.concatenate(idx_cols, axis=1)
        return carry

    lax.fori_loop(0, BLOCK // SLAB, body, 0)


def kernel(points, norms):
    pts = jnp.zeros((N, PAD_D), dtype=jnp.float32).at[:, :3].set(points)
    radii2d, idx = pl.pallas_call(
        _knn_block_kernel,
        grid=(N // BLOCK,),
        in_specs=[
            pl.BlockSpec((BLOCK, PAD_D), lambda i: (i, 0)),
            pl.BlockSpec((N, PAD_D), lambda i: (0, 0)),
        ],
        out_specs=[
            pl.BlockSpec((BLOCK, 1), lambda i: (i, 0)),
            pl.BlockSpec((BLOCK, K + 1), lambda i: (i, 0)),
        ],
        out_shape=[
            jax.ShapeDtypeStruct((N, 1), jnp.float32),
            jax.ShapeDtypeStruct((N, K + 1), jnp.int32),
        ],
        scratch_shapes=[pltpu.VMEM((BLOCK, N), jnp.float32)],
    )(pts, pts)
    radii = radii2d[:, 0]
    src = jnp.repeat(jnp.arange(N, dtype=jnp.int32), K)
    dst = idx[:, 1:].reshape(-1)
    edge_index = jnp.stack([src, dst], axis=0)
    return points, norms, radii, edge_index


# speedup vs baseline: 2.8206x; 2.8206x over previous
import jax
import jax.numpy as jnp
from jax import lax
from jax.experimental import pallas as pl
from jax.experimental.pallas import tpu as pltpu

N = 8192
K = 8
BLOCK = 256
SLAB = 32
PAD_D = 8
LANES = 128
NCOL = N // LANES


def _cmp_full(av, ai, bv, bi):
    if av is None:
        return bv, bi, av, ai
    if bv is None:
        return av, ai, bv, bi
    c = av <= bv
    return (jnp.minimum(av, bv), jnp.where(c, ai, bi),
            jnp.maximum(av, bv), jnp.where(c, bi, ai))


def _cmp_lo(av, ai, bv, bi):
    if av is None:
        return bv, bi
    if bv is None:
        return av, ai
    c = av <= bv
    return jnp.minimum(av, bv), jnp.where(c, ai, bi)


def _sort_bitonic(vals, idxs, need):
    n = len(vals)
    if n == 1:
        return vals, idxs
    half = n // 2
    lov = [None] * half
    loi = [None] * half
    if need > half:
        hiv = [None] * half
        hii = [None] * half
        for i in range(half):
            lov[i], loi[i], hiv[i], hii[i] = _cmp_full(
                vals[i], idxs[i], vals[i + half], idxs[i + half])
        sl_v, sl_i = _sort_bitonic(lov, loi, half)
        sh_v, sh_i = _sort_bitonic(hiv, hii, need - half)
        return sl_v + sh_v, sl_i + sh_i
    for i in range(half):
        lov[i], loi[i] = _cmp_lo(vals[i], idxs[i], vals[i + half], idxs[i + half])
    return _sort_bitonic(lov, loi, need)


def _merge_sorted(av, ai, bv, bi, keep):
    tot = len(av) + len(bv)
    p = 1
    while p < tot:
        p *= 2
    pad = p - tot
    seq_v = list(av) + [None] * pad + list(bv[::-1])
    seq_i = list(ai) + [None] * pad + list(bi[::-1])
    need = min(keep, tot)
    rv, ri = _sort_bitonic(seq_v, seq_i, need)
    return rv[:need], ri[:need]


def _tree_top9(load_col, lane):
    def build(lo, hi):
        if hi - lo == 1:
            return [load_col(lo)], [lane + lo * LANES]
        mid = (lo + hi) // 2
        av, ai = build(lo, mid)
        bv, bi = build(mid, hi)
        return _merge_sorted(av, ai, bv, bi, K + 1)
    return build(0, NCOL)


def _knn_block_kernel(q_ref, k_ref, radii_ref, idx_ref, d2_ref):
    q = q_ref[...]
    kt = k_ref[...]
    qq = jnp.sum(q * q, axis=1, keepdims=True)
    kk = jnp.sum(kt * kt, axis=1)[None, :]
    qk = lax.dot_general(q, kt, (((1,), (1,)), ((), ())),
                         preferred_element_type=jnp.float32)
    d2_ref[...] = jnp.maximum(qq + kk - 2.0 * qk, 0.0)

    lane = lax.broadcasted_iota(jnp.int32, (SLAB, LANES), 1)

    def body(i, carry):
        r0 = i * SLAB

        def load_col(c):
            return d2_ref[pl.ds(r0, SLAB), pl.ds(c * LANES, LANES)]

        lv, li = _tree_top9(load_col, lane)
        lv = list(lv)

        radii_acc = jnp.zeros((SLAB, 1), dtype=jnp.float32)
        idx_cols = []
        for r in range(K + 1):
            mv = lv[0]
            for t in range(1, r + 1):
                mv = jnp.minimum(mv, lv[t])
            m = jnp.min(mv, axis=1, keepdims=True)
            ai = jnp.full((SLAB, LANES), N, dtype=jnp.int32)
            for t in range(r + 1):
                ai = jnp.where(lv[t] == m, jnp.minimum(ai, li[t]), ai)
            am = jnp.min(ai, axis=1, keepdims=True)
            idx_cols.append(am)
            if r > 0:
                radii_acc = radii_acc + jnp.sqrt(jnp.maximum(m, 1e-12))
            if r < K:
                for t in range(r + 1):
                    lv[t] = jnp.where(li[t] == am, jnp.float32(jnp.inf), lv[t])
        radii_ref[pl.ds(r0, SLAB), :] = radii_acc * (1.0 / K)
        idx_ref[pl.ds(r0, SLAB), :] = jnp.concatenate(idx_cols, axis=1)
        return carry

    lax.fori_loop(0, BLOCK // SLAB, body, 0)


def kernel(points, norms):
    pts = jnp.zeros((N, PAD_D), dtype=jnp.float32).at[:, :3].set(points)
    radii2d, idx = pl.pallas_call(
        _knn_block_kernel,
        grid=(N // BLOCK,),
        in_specs=[
            pl.BlockSpec((BLOCK, PAD_D), lambda i: (i, 0)),
            pl.BlockSpec((N, PAD_D), lambda i: (0, 0)),
        ],
        out_specs=[
            pl.BlockSpec((BLOCK, 1), lambda i: (i, 0)),
            pl.BlockSpec((BLOCK, K + 1), lambda i: (i, 0)),
        ],
        out_shape=[
            jax.ShapeDtypeStruct((N, 1), jnp.float32),
            jax.ShapeDtypeStruct((N, K + 1), jnp.int32),
        ],
        scratch_shapes=[pltpu.VMEM((BLOCK, N), jnp.float32)],
    )(pts, pts)
    radii = radii2d[:, 0]
    src = jnp.repeat(jnp.arange(N, dtype=jnp.int32), K)
    dst = idx[:, 1:].reshape(-1)
    edge_index = jnp.stack([src, dst], axis=0)
    return points, norms, radii, edge_index


# speedup vs baseline: 5.7085x; 2.0239x over previous
import jax
import jax.numpy as jnp
from jax import lax
from jax.experimental import pallas as pl

N = 8192
K = 8
BLOCK = 256
PAD_D = 8
LANES = 128
NCOL = N // LANES


def _cmp_full(av, ai, bv, bi):
    if av is None:
        return bv, bi, av, ai
    if bv is None:
        return av, ai, bv, bi
    c = av <= bv
    return (jnp.minimum(av, bv), jnp.where(c, ai, bi),
            jnp.maximum(av, bv), jnp.where(c, bi, ai))


def _cmp_lo(av, ai, bv, bi):
    if av is None:
        return bv, bi
    if bv is None:
        return av, ai
    c = av <= bv
    return jnp.minimum(av, bv), jnp.where(c, ai, bi)


def _sort_bitonic(vals, idxs, need):
    n = len(vals)
    if n == 1:
        return vals, idxs
    half = n // 2
    lov = [None] * half
    loi = [None] * half
    if need > half:
        hiv = [None] * half
        hii = [None] * half
        for i in range(half):
            lov[i], loi[i], hiv[i], hii[i] = _cmp_full(
                vals[i], idxs[i], vals[i + half], idxs[i + half])
        sl_v, sl_i = _sort_bitonic(lov, loi, half)
        sh_v, sh_i = _sort_bitonic(hiv, hii, need - half)
        return sl_v + sh_v, sl_i + sh_i
    for i in range(half):
        lov[i], loi[i] = _cmp_lo(vals[i], idxs[i], vals[i + half], idxs[i + half])
    return _sort_bitonic(lov, loi, need)


def _merge_sorted(av, ai, bv, bi, keep):
    tot = len(av) + len(bv)
    p = 1
    while p < tot:
        p *= 2
    pad = p - tot
    seq_v = list(av) + [None] * pad + list(bv[::-1])
    seq_i = list(ai) + [None] * pad + list(bi[::-1])
    need = min(keep, tot)
    rv, ri = _sort_bitonic(seq_v, seq_i, need)
    return rv[:need], ri[:need]


def _knn_block_kernel(q_ref, k_ref, radii_ref, idx_ref):
    qm = q_ref[...]
    km = k_ref[...]
    score = lax.dot_general(qm, km, (((1,), (1,)), ((), ())),
                            preferred_element_type=jnp.float32)
    qq = jnp.sum(qm[:, 0:3] * qm[:, 0:3], axis=1, keepdims=True) * 0.25

    lane = lax.broadcasted_iota(jnp.int32, (BLOCK, LANES), 1)
    groups = [([score[:, c * LANES:(c + 1) * LANES]], [lane + c * LANES])
              for c in range(NCOL)]
    while len(groups) > 1:
        nxt = []
        for g in range(0, len(groups), 2):
            (av, ai), (bv, bi) = groups[g], groups[g + 1]
            nxt.append(_merge_sorted(av, ai, bv, bi, K + 1))
        groups = nxt
    lv, li = groups[0]
    lv = list(lv)

    radii_acc = jnp.zeros((BLOCK, 1), dtype=jnp.float32)
    idx_cols = []
    for r in range(K + 1):
        mv = lv[0]
        for t in range(1, r + 1):
            mv = jnp.minimum(mv, lv[t])
        m = jnp.min(mv, axis=1, keepdims=True)
        ai = jnp.full((BLOCK, LANES), N, dtype=jnp.int32)
        for t in range(r + 1):
            ai = jnp.where(lv[t] == m, jnp.minimum(ai, li[t]), ai)
        am = jnp.min(ai, axis=1, keepdims=True)
        idx_cols.append(am)
        if r > 0:
            radii_acc = radii_acc + jnp.sqrt(jnp.maximum(m + qq, 1e-12))
        if r < K:
            for t in range(r + 1):
                lv[t] = jnp.where(li[t] == am, jnp.float32(jnp.inf), lv[t])
    idx_ref[...] = jnp.concatenate(idx_cols, axis=1)
    radii_ref[...] = radii_acc * (1.0 / K)


def kernel(points, norms):
    qm = jnp.zeros((N, PAD_D), dtype=jnp.float32)
    qm = qm.at[:, 0:3].set(points * -2.0).at[:, 3].set(1.0)
    km = jnp.zeros((N, PAD_D), dtype=jnp.float32)
    km = km.at[:, 0:3].set(points).at[:, 3].set(jnp.sum(points * points, axis=1))
    radii2d, idx = pl.pallas_call(
        _knn_block_kernel,
        grid=(N // BLOCK,),
        in_specs=[
            pl.BlockSpec((BLOCK, PAD_D), lambda i: (i, 0)),
            pl.BlockSpec((N, PAD_D), lambda i: (0, 0)),
        ],
        out_specs=[
            pl.BlockSpec((BLOCK, 1), lambda i: (i, 0)),
            pl.BlockSpec((BLOCK, K + 1), lambda i: (i, 0)),
        ],
        out_shape=[
            jax.ShapeDtypeStruct((N, 1), jnp.float32),
            jax.ShapeDtypeStruct((N, K + 1), jnp.int32),
        ],
    )(qm, km)
    radii = radii2d[:, 0]
    src = jnp.repeat(jnp.arange(N, dtype=jnp.int32), K)
    dst = idx[:, 1:].reshape(-1)
    edge_index = jnp.stack([src, dst], axis=0)
    return points, norms, radii, edge_index
